# baseline (speedup 1.0000x reference)
import functools

import jax
import jax.numpy as jnp
from jax import lax
from jax.experimental import pallas as pl
from jax.experimental.pallas import tpu as pltpu
from jax.experimental.pallas import tpu_sc as plsc

EMB = 50
BATCH = 16384
NC, NS, L = 2, 16, 16
NW = NC * NS
BPW = BATCH // NW
CHUNKS = BPW // L
IG = 128
NG = BPW // IG


def _sc_body(gidx_hbm, pidx_hbm, emb1_hbm, emb2_hbm, b1_hbm, b2_hbm,
             out_hbm, gidx_v, pidx_v, g_v, p_v, b1_v, b2_v, out_v, sem):
    wid = lax.axis_index("s") * NC + lax.axis_index("c")
    base = wid * BPW

    for k in range(NG):
        pltpu.sync_copy(gidx_hbm.at[pl.ds(base + k * IG, IG)], gidx_v.at[k])
        pltpu.sync_copy(pidx_hbm.at[pl.ds(base + k * IG, IG)], pidx_v.at[k])

    copies = []
    for k in range(NG):
        sl = pl.ds(k * IG, IG)
        copies.append(pltpu.async_copy(
            emb1_hbm.at[gidx_v.at[k]], g_v.at[sl], sem))
        copies.append(pltpu.async_copy(
            emb2_hbm.at[pidx_v.at[k]], p_v.at[sl], sem))
        copies.append(pltpu.async_copy(
            b1_hbm.at[gidx_v.at[k]], b1_v.at[sl], sem))
        copies.append(pltpu.async_copy(
            b2_hbm.at[pidx_v.at[k]], b2_v.at[sl], sem))
    for cp in copies:
        cp.wait()

    lanes = lax.iota(jnp.int32, L)

    def chunk(c, carry):
        r0 = c * L
        rows = r0 + lanes
        acc = b1_v[pl.ds(r0, L)] + b2_v[pl.ds(r0, L)]
        for j in range(EMB):
            jv = jnp.full((L,), j, jnp.int32)
            gv = plsc.load_gather(g_v, [rows, jv])
            pv = plsc.load_gather(p_v, [rows, jv])
            acc = acc + gv * pv
        out_v[pl.ds(r0, L)] = acc
        return carry

    lax.fori_loop(0, CHUNKS, chunk, 0)

    pltpu.sync_copy(out_v, out_hbm.at[pl.ds(base, BPW)])


def _sc_call(gidx, pidx, emb_1, emb_2, b1, b2):
    mesh = plsc.VectorSubcoreMesh(core_axis_name="c", subcore_axis_name="s")
    k = functools.partial(
        pl.kernel,
        mesh=mesh,
        out_type=jax.ShapeDtypeStruct((BATCH,), jnp.float32),
        scratch_types=[
            pltpu.VMEM((NG, IG), jnp.int32),
            pltpu.VMEM((NG, IG), jnp.int32),
            pltpu.VMEM((BPW, EMB), jnp.float32),
            pltpu.VMEM((BPW, EMB), jnp.float32),
            pltpu.VMEM((BPW,), jnp.float32),
            pltpu.VMEM((BPW,), jnp.float32),
            pltpu.VMEM((BPW,), jnp.float32),
            pltpu.SemaphoreType.DMA,
        ],
        compiler_params=pltpu.CompilerParams(
            needs_layout_passes=False, use_tc_tiling_on_sc=False),
    )(_sc_body)
    return k(gidx, pidx, emb_1, emb_2, b1, b2)


def kernel(x, emb_1, emb_2, emb_1_bias, emb_2_bias):
    gidx = x[:, 0].astype(jnp.int32)
    pidx = x[:, 1].astype(jnp.int32)
    b1 = emb_1_bias.reshape(-1)
    b2 = emb_2_bias.reshape(-1)
    return _sc_call(gidx, pidx, emb_1, emb_2, b1, b2)


# speedup vs baseline: 3.2378x; 3.2378x over previous
import functools

import jax
import jax.numpy as jnp
from jax import lax
from jax.experimental import pallas as pl
from jax.experimental.pallas import tpu as pltpu
from jax.experimental.pallas import tpu_sc as plsc

EMB = 50
BATCH = 16384
NC, NS, L = 2, 16, 16
NW = NC * NS
BPW = BATCH // NW
HALF = BPW // 2
HCHUNKS = HALF // L
IG = 128
NG = BPW // IG


def _sc_body(gidx_hbm, pidx_hbm, emb1_hbm, emb2_hbm, b1_hbm, b2_hbm,
             out_hbm, gidx_v, pidx_v, g_v, p_v, b1_v, b2_v,
             out_v, sem, rsem):
    wid = lax.axis_index("s") * NC + lax.axis_index("c")
    base = wid * BPW

    for k in range(NG):
        pltpu.sync_copy(gidx_hbm.at[pl.ds(base + k * IG, IG)], gidx_v.at[k])
        pltpu.sync_copy(pidx_hbm.at[pl.ds(base + k * IG, IG)], pidx_v.at[k])

    bias_copies = []
    for k in range(NG):
        sl = pl.ds(k * IG, IG)
        bias_copies.append(pltpu.async_copy(
            b1_hbm.at[gidx_v.at[k]], b1_v.at[sl], sem))
        bias_copies.append(pltpu.async_copy(
            b2_hbm.at[pidx_v.at[k]], b2_v.at[sl], sem))
    for cp in bias_copies:
        cp.wait()

    lanes = lax.iota(jnp.int32, L)

    def half(h, carry):
        hbase = h * HALF

        def fire(grp, c):
            k = (hbase // IG) + grp // (IG // L)
            i = (grp % (IG // L)) * L
            gv16 = gidx_v[k, pl.ds(i, L)]
            pv16 = pidx_v[k, pl.ds(i, L)]
            r0 = grp * L
            for t in range(L):
                pltpu.async_copy(
                    emb1_hbm.at[gv16[t]], g_v.at[r0 + t], rsem)
                pltpu.async_copy(
                    emb2_hbm.at[pv16[t]], p_v.at[r0 + t], rsem)
            return c

        lax.fori_loop(0, HALF // L, fire, 0)

        pltpu.make_async_copy(
            emb1_hbm.at[pl.ds(0, HALF)], g_v, rsem).wait()
        pltpu.make_async_copy(
            emb2_hbm.at[pl.ds(0, HALF)], p_v, rsem).wait()

        def chunk(c, carry2):
            r0 = c * L
            rows = r0 + lanes
            acc = (b1_v[pl.ds(hbase + r0, L)] + b2_v[pl.ds(hbase + r0, L)])
            for j in range(EMB):
                jv = jnp.full((L,), j, jnp.int32)
                gv = plsc.load_gather(g_v, [rows, jv])
                pv = plsc.load_gather(p_v, [rows, jv])
                acc = acc + gv * pv
            out_v[pl.ds(hbase + r0, L)] = acc
            return carry2

        lax.fori_loop(0, HCHUNKS, chunk, 0)
        return carry

    lax.fori_loop(0, 2, half, 0)

    pltpu.sync_copy(out_v, out_hbm.at[pl.ds(base, BPW)])


def _sc_call(gidx, pidx, emb_1, emb_2, b1, b2):
    mesh = plsc.VectorSubcoreMesh(core_axis_name="c", subcore_axis_name="s")
    k = functools.partial(
        pl.kernel,
        mesh=mesh,
        out_type=jax.ShapeDtypeStruct((BATCH,), jnp.float32),
        scratch_types=[
            pltpu.VMEM((NG, IG), jnp.int32),
            pltpu.VMEM((NG, IG), jnp.int32),
            pltpu.VMEM((HALF, EMB), jnp.float32),
            pltpu.VMEM((HALF, EMB), jnp.float32),
            pltpu.VMEM((BPW,), jnp.float32),
            pltpu.VMEM((BPW,), jnp.float32),
            pltpu.VMEM((BPW,), jnp.float32),
            pltpu.SemaphoreType.DMA,
            pltpu.SemaphoreType.DMA,
        ],
        compiler_params=pltpu.CompilerParams(needs_layout_passes=False),
    )(_sc_body)
    return k(gidx, pidx, emb_1, emb_2, b1, b2)


def kernel(x, emb_1, emb_2, emb_1_bias, emb_2_bias):
    gidx = x[:, 0].astype(jnp.int32)
    pidx = x[:, 1].astype(jnp.int32)
    b1 = emb_1_bias.reshape(-1)
    b2 = emb_2_bias.reshape(-1)
    return _sc_call(gidx, pidx, emb_1, emb_2, b1, b2)
